# baseline (speedup 1.0000x reference)
import functools

import jax
import jax.numpy as jnp
from jax import lax
from jax.experimental import pallas as pl
from jax.experimental.pallas import tpu as pltpu
from jax.experimental.pallas import tpu_sc as plsc

_B, _T, _H, _D, _K = 16, 576, 2048, 256, 8192
_N = _B * _T
_TB = 256
_NB = _N // _TB
_KC = 2048
_NKC = _K // _KC
_TB2 = 512
_NB2 = _N // _TB2

_NW = 32
_HALVES = 2


def _esq_body(cb_ref, esq_ref, cbm2_ref):
    c = cb_ref[...]
    esq_ref[...] = jnp.sum(c * c, axis=1, keepdims=True)
    cbm2_ref[...] = c.astype(jnp.bfloat16) * jnp.asarray(-2, jnp.bfloat16)


def _argmin_body(x_ref, w1_ref, b1_ref, cbm2_ref, esq_ref, idx_ref):
    f32 = jnp.float32
    bf16 = jnp.bfloat16
    z = jnp.dot(x_ref[...].astype(bf16), w1_ref[...],
                preferred_element_type=f32)
    z = z + b1_ref[...]
    zsq = jnp.sum(z * z, axis=1, keepdims=True)
    zbf = z.astype(bf16)
    best_d = None
    best_i = None
    for c in range(_NKC):
        m2 = jax.lax.dot_general(
            zbf, cbm2_ref[c * _KC:(c + 1) * _KC, :],
            dimension_numbers=(((1,), (1,)), ((), ())),
            preferred_element_type=f32)
        d = (zsq + m2) + esq_ref[:, c * _KC:(c + 1) * _KC]
        dmin = jnp.min(d, axis=1, keepdims=True)
        lane = jax.lax.broadcasted_iota(jnp.int32, (_TB, _KC), 1).astype(f32)
        li = jnp.min(jnp.where(d == dmin, lane, f32(2 * _K)), axis=1)
        li = li + f32(c * _KC)
        if c == 0:
            best_d = dmin[:, 0]
            best_i = li
        else:
            better = dmin[:, 0] < best_d
            best_i = jnp.where(better, li, best_i)
            best_d = jnp.minimum(dmin[:, 0], best_d)
    idx_ref[0, 0, :] = best_i.astype(jnp.int32)


def _decode_body(q_ref, w2_ref, b2_ref, out_ref):
    q = q_ref[...].astype(jnp.bfloat16)
    out = jnp.dot(q, w2_ref[...], preferred_element_type=jnp.float32)
    out_ref[...] = out + b2_ref[...]


def _sc_gather_body(table_hbm, idx_hbm, out_hbm, idx_v, rows_v, sem, *, bpw):
    wid = lax.axis_index("s") * 2 + lax.axis_index("c")
    base = wid * bpw
    pltpu.sync_copy(idx_hbm.at[pl.ds(base, bpw)], idx_v)
    pltpu.async_copy(table_hbm.at[idx_v], rows_v, sem).wait()
    pltpu.sync_copy(rows_v, out_hbm.at[pl.ds(base, bpw)])


@jax.jit
def kernel(image_features, W1, b1, codebook, W2, b2):
    f32 = jnp.float32
    bf16 = jnp.bfloat16
    x = image_features.reshape(_N, _H)
    e_sq, cbm2 = pl.pallas_call(
        _esq_body,
        out_shape=[
            jax.ShapeDtypeStruct((_K, 1), f32),
            jax.ShapeDtypeStruct((_K, _D), bf16),
        ],
    )(codebook)
    e_sq = e_sq.reshape(1, _K)

    w1bf = W1.astype(bf16)
    w2bf = W2.astype(bf16)
    b1r = b1.reshape(1, _D)
    b2r = b2.reshape(1, _H)

    nh = _N // _HALVES
    nbh = nh // _TB
    nb2h = nh // _TB2
    bpwh = nh // _NW

    gather = functools.partial(
        pl.kernel,
        mesh=plsc.VectorSubcoreMesh(core_axis_name="c", subcore_axis_name="s"),
        out_type=jax.ShapeDtypeStruct((nh, _D), f32),
        scratch_types=[
            pltpu.VMEM((bpwh,), jnp.int32),
            pltpu.VMEM((bpwh, _D), f32),
            pltpu.SemaphoreType.DMA,
        ],
    )(functools.partial(_sc_gather_body, bpw=bpwh))

    idx_halves = []
    out_halves = []
    for h in range(_HALVES):
        idx3 = pl.pallas_call(
            _argmin_body,
            grid=(nbh,),
            in_specs=[
                pl.BlockSpec((_TB, _H), lambda i: (i, 0)),
                pl.BlockSpec((_H, _D), lambda i: (0, 0)),
                pl.BlockSpec((1, _D), lambda i: (0, 0)),
                pl.BlockSpec((_K, _D), lambda i: (0, 0)),
                pl.BlockSpec((1, _K), lambda i: (0, 0)),
            ],
            out_specs=pl.BlockSpec((1, 1, _TB), lambda i: (i, 0, 0)),
            out_shape=jax.ShapeDtypeStruct((nbh, 1, _TB), jnp.int32),
        )(x[h * nh:(h + 1) * nh], w1bf, b1r, cbm2, e_sq)
        idx_halves.append(idx3.reshape(nh))

    for h in range(_HALVES):
        q = gather(codebook, idx_halves[h])
        out = pl.pallas_call(
            _decode_body,
            grid=(nb2h,),
            in_specs=[
                pl.BlockSpec((_TB2, _D), lambda i: (i, 0)),
                pl.BlockSpec((_D, _H), lambda i: (0, 0)),
                pl.BlockSpec((1, _H), lambda i: (0, 0)),
            ],
            out_specs=pl.BlockSpec((_TB2, _H), lambda i: (i, 0)),
            out_shape=jax.ShapeDtypeStruct((nh, _H), f32),
        )(q, w2bf, b2r)
        out_halves.append(out)

    out = jnp.concatenate(out_halves, axis=0)
    idx = jnp.concatenate(idx_halves, axis=0)
    return out.reshape(_B, _T, _H), idx.reshape(_B, _T)


# speedup vs baseline: 1.5916x; 1.5916x over previous
import functools

import jax
import jax.numpy as jnp
from jax import lax
from jax.experimental import pallas as pl
from jax.experimental.pallas import tpu as pltpu
from jax.experimental.pallas import tpu_sc as plsc

_B, _T, _H, _D, _K = 16, 576, 2048, 256, 8192
_N = _B * _T
_TB = 512
_NB = _N // _TB
_KC = 2048
_NKC = _K // _KC
_TB2 = 512
_NB2 = _N // _TB2

_NW = 32
_BPW = _N // _NW


def _esq_body(cb_ref, esq_ref, cbm2_ref):
    c = cb_ref[...]
    esq_ref[...] = jnp.sum(c * c, axis=1, keepdims=True)
    cbm2_ref[...] = c.astype(jnp.bfloat16) * jnp.asarray(-2, jnp.bfloat16)


def _argmin_body(x_ref, w1_ref, b1_ref, cbm2_ref, esq_ref, idx_ref):
    f32 = jnp.float32
    bf16 = jnp.bfloat16
    z = jnp.dot(x_ref[...].astype(bf16), w1_ref[...],
                preferred_element_type=f32)
    z = z + b1_ref[...]
    zsq = jnp.sum(z * z, axis=1, keepdims=True)
    zbf = z.astype(bf16)
    best_d = None
    best_i = None
    for c in range(_NKC):
        m2 = jax.lax.dot_general(
            zbf, cbm2_ref[c * _KC:(c + 1) * _KC, :],
            dimension_numbers=(((1,), (1,)), ((), ())),
            preferred_element_type=f32)
        d = (zsq + m2) + esq_ref[:, c * _KC:(c + 1) * _KC]
        dmin = jnp.min(d, axis=1, keepdims=True)
        lane = jax.lax.broadcasted_iota(jnp.int32, (_TB, _KC), 1).astype(f32)
        li = jnp.min(jnp.where(d == dmin, lane, f32(2 * _K)), axis=1)
        li = li + f32(c * _KC)
        if c == 0:
            best_d = dmin[:, 0]
            best_i = li
        else:
            better = dmin[:, 0] < best_d
            best_i = jnp.where(better, li, best_i)
            best_d = jnp.minimum(dmin[:, 0], best_d)
    idx_ref[0, 0, :] = best_i.astype(jnp.int32)


def _decode_body(q_ref, w2_ref, b2_ref, out_ref):
    q = q_ref[...].astype(jnp.bfloat16)
    out = jnp.dot(q, w2_ref[...], preferred_element_type=jnp.float32)
    out_ref[...] = out + b2_ref[...]


def _sc_gather_body(table_hbm, idx_hbm, out_hbm, idx_v, rows_v, sem):
    wid = lax.axis_index("s") * 2 + lax.axis_index("c")
    base = wid * _BPW
    pltpu.sync_copy(idx_hbm.at[pl.ds(base, _BPW)], idx_v)
    pltpu.async_copy(table_hbm.at[idx_v], rows_v, sem).wait()
    pltpu.sync_copy(rows_v, out_hbm.at[pl.ds(base, _BPW)])


@jax.jit
def kernel(image_features, W1, b1, codebook, W2, b2):
    f32 = jnp.float32
    bf16 = jnp.bfloat16
    x = image_features.reshape(_N, _H)
    e_sq, cbm2 = pl.pallas_call(
        _esq_body,
        out_shape=[
            jax.ShapeDtypeStruct((_K, 1), f32),
            jax.ShapeDtypeStruct((_K, _D), bf16),
        ],
    )(codebook)
    e_sq = e_sq.reshape(1, _K)

    idx3 = pl.pallas_call(
        _argmin_body,
        grid=(_NB,),
        in_specs=[
            pl.BlockSpec((_TB, _H), lambda i: (i, 0)),
            pl.BlockSpec((_H, _D), lambda i: (0, 0)),
            pl.BlockSpec((1, _D), lambda i: (0, 0)),
            pl.BlockSpec((_K, _D), lambda i: (0, 0)),
            pl.BlockSpec((1, _K), lambda i: (0, 0)),
        ],
        out_specs=pl.BlockSpec((1, 1, _TB), lambda i: (i, 0, 0)),
        out_shape=jax.ShapeDtypeStruct((_NB, 1, _TB), jnp.int32),
    )(x, W1.astype(bf16), b1.reshape(1, _D), cbm2, e_sq)
    idx = idx3.reshape(_N)

    gather = functools.partial(
        pl.kernel,
        mesh=plsc.VectorSubcoreMesh(core_axis_name="c", subcore_axis_name="s"),
        out_type=jax.ShapeDtypeStruct((_N, _D), f32),
        scratch_types=[
            pltpu.VMEM((_BPW,), jnp.int32),
            pltpu.VMEM((_BPW, _D), f32),
            pltpu.SemaphoreType.DMA,
        ],
    )(_sc_gather_body)
    q = gather(codebook, idx)

    out = pl.pallas_call(
        _decode_body,
        grid=(_NB2,),
        in_specs=[
            pl.BlockSpec((_TB2, _D), lambda i: (i, 0)),
            pl.BlockSpec((_D, _H), lambda i: (0, 0)),
            pl.BlockSpec((1, _H), lambda i: (0, 0)),
        ],
        out_specs=pl.BlockSpec((_TB2, _H), lambda i: (i, 0)),
        out_shape=jax.ShapeDtypeStruct((_N, _H), f32),
    )(q, W2.astype(bf16), b2.reshape(1, _H))
    return out.reshape(_B, _T, _H), idx.reshape(_B, _T)


# speedup vs baseline: 1.6379x; 1.0291x over previous
import functools

import jax
import jax.numpy as jnp
from jax import lax
from jax.experimental import pallas as pl
from jax.experimental.pallas import tpu as pltpu
from jax.experimental.pallas import tpu_sc as plsc

_B, _T, _H, _D, _K = 16, 576, 2048, 256, 8192
_N = _B * _T
_TB = 512
_NB = _N // _TB
_KC = 1024
_NKC = _K // _KC
_TB2 = 512
_NB2 = _N // _TB2

_NW = 32
_BPW = _N // _NW


def _esq_body(cb_ref, esq_ref, cbm2_ref):
    c = cb_ref[...]
    esq_ref[...] = jnp.sum(c * c, axis=1, keepdims=True)
    cbm2_ref[...] = c.astype(jnp.bfloat16) * jnp.asarray(-2, jnp.bfloat16)


def _argmin_body(x_ref, w1_ref, b1_ref, cbm2_ref, esq_ref, idx_ref):
    f32 = jnp.float32
    bf16 = jnp.bfloat16
    z = jnp.dot(x_ref[...].astype(bf16), w1_ref[...],
                preferred_element_type=f32)
    z = z + b1_ref[...]
    zsq = jnp.sum(z * z, axis=1, keepdims=True)
    zbf = z.astype(bf16)
    best_d = None
    best_i = None
    for c in range(_NKC):
        m2 = jax.lax.dot_general(
            zbf, cbm2_ref[c * _KC:(c + 1) * _KC, :],
            dimension_numbers=(((1,), (1,)), ((), ())),
            preferred_element_type=f32)
        d = (zsq + m2) + esq_ref[:, c * _KC:(c + 1) * _KC]
        dmin = jnp.min(d, axis=1, keepdims=True)
        lane = jax.lax.broadcasted_iota(jnp.int32, (_TB, _KC), 1).astype(f32)
        li = jnp.min(jnp.where(d == dmin, lane, f32(2 * _K)), axis=1)
        li = li + f32(c * _KC)
        if c == 0:
            best_d = dmin[:, 0]
            best_i = li
        else:
            better = dmin[:, 0] < best_d
            best_i = jnp.where(better, li, best_i)
            best_d = jnp.minimum(dmin[:, 0], best_d)
    idx_ref[0, 0, :] = best_i.astype(jnp.int32)


def _decode_body(q_ref, w2_ref, b2_ref, out_ref):
    q = q_ref[...].astype(jnp.bfloat16)
    out = jnp.dot(q, w2_ref[...], preferred_element_type=jnp.float32)
    out_ref[...] = out + b2_ref[...]


def _sc_gather_body(table_hbm, idx_hbm, out_hbm, idx_v, rows_v, sem):
    wid = lax.axis_index("s") * 2 + lax.axis_index("c")
    base = wid * _BPW
    pltpu.sync_copy(idx_hbm.at[pl.ds(base, _BPW)], idx_v)
    pltpu.async_copy(table_hbm.at[idx_v], rows_v, sem).wait()
    pltpu.sync_copy(rows_v, out_hbm.at[pl.ds(base, _BPW)])


@jax.jit
def kernel(image_features, W1, b1, codebook, W2, b2):
    f32 = jnp.float32
    bf16 = jnp.bfloat16
    x = image_features.reshape(_N, _H)
    e_sq, cbm2 = pl.pallas_call(
        _esq_body,
        out_shape=[
            jax.ShapeDtypeStruct((_K, 1), f32),
            jax.ShapeDtypeStruct((_K, _D), bf16),
        ],
    )(codebook)
    e_sq = e_sq.reshape(1, _K)

    idx3 = pl.pallas_call(
        _argmin_body,
        grid=(_NB,),
        in_specs=[
            pl.BlockSpec((_TB, _H), lambda i: (i, 0)),
            pl.BlockSpec((_H, _D), lambda i: (0, 0)),
            pl.BlockSpec((1, _D), lambda i: (0, 0)),
            pl.BlockSpec((_K, _D), lambda i: (0, 0)),
            pl.BlockSpec((1, _K), lambda i: (0, 0)),
        ],
        out_specs=pl.BlockSpec((1, 1, _TB), lambda i: (i, 0, 0)),
        out_shape=jax.ShapeDtypeStruct((_NB, 1, _TB), jnp.int32),
    )(x, W1.astype(bf16), b1.reshape(1, _D), cbm2, e_sq)
    idx = idx3.reshape(_N)

    gather = functools.partial(
        pl.kernel,
        mesh=plsc.VectorSubcoreMesh(core_axis_name="c", subcore_axis_name="s"),
        out_type=jax.ShapeDtypeStruct((_N, _D), f32),
        scratch_types=[
            pltpu.VMEM((_BPW,), jnp.int32),
            pltpu.VMEM((_BPW, _D), f32),
            pltpu.SemaphoreType.DMA,
        ],
    )(_sc_gather_body)
    q = gather(codebook, idx)

    out = pl.pallas_call(
        _decode_body,
        grid=(_NB2,),
        in_specs=[
            pl.BlockSpec((_TB2, _D), lambda i: (i, 0)),
            pl.BlockSpec((_D, _H), lambda i: (0, 0)),
            pl.BlockSpec((1, _H), lambda i: (0, 0)),
        ],
        out_specs=pl.BlockSpec((_TB2, _H), lambda i: (i, 0)),
        out_shape=jax.ShapeDtypeStruct((_N, _H), f32),
    )(q, W2.astype(bf16), b2.reshape(1, _H))
    return out.reshape(_B, _T, _H), idx.reshape(_B, _T)


# speedup vs baseline: 1.6621x; 1.0148x over previous
import functools

import jax
import jax.numpy as jnp
from jax import lax
from jax.experimental import pallas as pl
from jax.experimental.pallas import tpu as pltpu
from jax.experimental.pallas import tpu_sc as plsc

_B, _T, _H, _D, _K = 16, 576, 2048, 256, 8192
_N = _B * _T
_TB = 512
_NB = _N // _TB
_KC = 1024
_NKC = _K // _KC
_TB2 = 512
_NB2 = _N // _TB2

_NW = 32
_BPW = _N // _NW


def _prep_body(cb_ref, w1_ref, w2_ref, esq_ref, cbm2_ref, w1bf_ref, w2bf_ref):
    c = cb_ref[...]
    s = jnp.sum(c * c, axis=1, keepdims=True)
    esq_ref[...] = s.reshape(1, _K)
    cbm2_ref[...] = c.astype(jnp.bfloat16) * jnp.asarray(-2, jnp.bfloat16)
    w1bf_ref[...] = w1_ref[...].astype(jnp.bfloat16)
    w2bf_ref[...] = w2_ref[...].astype(jnp.bfloat16)


def _argmin_body(x_ref, w1_ref, b1_ref, cbm2_ref, esq_ref, idx_ref):
    f32 = jnp.float32
    bf16 = jnp.bfloat16
    z = jnp.dot(x_ref[...].astype(bf16), w1_ref[...],
                preferred_element_type=f32)
    z = z + b1_ref[...]
    zsq = jnp.sum(z * z, axis=1, keepdims=True)
    zbf = z.astype(bf16)
    best_d = None
    best_i = None
    for c in range(_NKC):
        m2 = jax.lax.dot_general(
            zbf, cbm2_ref[c * _KC:(c + 1) * _KC, :],
            dimension_numbers=(((1,), (1,)), ((), ())),
            preferred_element_type=f32)
        d = (zsq + m2) + esq_ref[:, c * _KC:(c + 1) * _KC]
        dmin = jnp.min(d, axis=1, keepdims=True)
        lane = jax.lax.broadcasted_iota(jnp.int32, (_TB, _KC), 1).astype(f32)
        li = jnp.min(jnp.where(d == dmin, lane, f32(2 * _K)), axis=1)
        li = li + f32(c * _KC)
        if c == 0:
            best_d = dmin[:, 0]
            best_i = li
        else:
            better = dmin[:, 0] < best_d
            best_i = jnp.where(better, li, best_i)
            best_d = jnp.minimum(dmin[:, 0], best_d)
    idx_ref[0, 0, :] = best_i.astype(jnp.int32)


def _decode_body(q_ref, w2_ref, b2_ref, out_ref):
    q = q_ref[...].astype(jnp.bfloat16)
    out = jnp.dot(q, w2_ref[...], preferred_element_type=jnp.float32)
    out_ref[...] = out + b2_ref[...]


def _sc_gather_body(table_hbm, idx_hbm, out_hbm, idx_v, rows_v, sem):
    wid = lax.axis_index("s") * 2 + lax.axis_index("c")
    base = wid * _BPW
    pltpu.sync_copy(idx_hbm.at[pl.ds(base, _BPW)], idx_v)
    pltpu.async_copy(table_hbm.at[idx_v], rows_v, sem).wait()
    pltpu.sync_copy(rows_v, out_hbm.at[pl.ds(base, _BPW)])


@jax.jit
def kernel(image_features, W1, b1, codebook, W2, b2):
    f32 = jnp.float32
    bf16 = jnp.bfloat16
    x = image_features.reshape(_N, _H)
    e_sq, cbm2, w1bf, w2bf = pl.pallas_call(
        _prep_body,
        out_shape=[
            jax.ShapeDtypeStruct((1, _K), f32),
            jax.ShapeDtypeStruct((_K, _D), bf16),
            jax.ShapeDtypeStruct((_H, _D), bf16),
            jax.ShapeDtypeStruct((_D, _H), bf16),
        ],
    )(codebook, W1, W2)

    idx3 = pl.pallas_call(
        _argmin_body,
        grid=(_NB,),
        in_specs=[
            pl.BlockSpec((_TB, _H), lambda i: (i, 0)),
            pl.BlockSpec((_H, _D), lambda i: (0, 0)),
            pl.BlockSpec((1, _D), lambda i: (0, 0)),
            pl.BlockSpec((_K, _D), lambda i: (0, 0)),
            pl.BlockSpec((1, _K), lambda i: (0, 0)),
        ],
        out_specs=pl.BlockSpec((1, 1, _TB), lambda i: (i, 0, 0)),
        out_shape=jax.ShapeDtypeStruct((_NB, 1, _TB), jnp.int32),
    )(x, w1bf, b1.reshape(1, _D), cbm2, e_sq)
    idx = idx3.reshape(_N)

    gather = functools.partial(
        pl.kernel,
        mesh=plsc.VectorSubcoreMesh(core_axis_name="c", subcore_axis_name="s"),
        out_type=jax.ShapeDtypeStruct((_N, _D), f32),
        scratch_types=[
            pltpu.VMEM((_BPW,), jnp.int32),
            pltpu.VMEM((_BPW, _D), f32),
            pltpu.SemaphoreType.DMA,
        ],
    )(_sc_gather_body)
    q = gather(codebook, idx)

    out = pl.pallas_call(
        _decode_body,
        grid=(_NB2,),
        in_specs=[
            pl.BlockSpec((_TB2, _D), lambda i: (i, 0)),
            pl.BlockSpec((_D, _H), lambda i: (0, 0)),
            pl.BlockSpec((1, _H), lambda i: (0, 0)),
        ],
        out_specs=pl.BlockSpec((_TB2, _H), lambda i: (i, 0)),
        out_shape=jax.ShapeDtypeStruct((_N, _H), f32),
    )(q, w2bf, b2.reshape(1, _H))
    return out.reshape(_B, _T, _H), idx.reshape(_B, _T)
